# baseline (speedup 1.0000x reference)
import functools

import jax
import jax.numpy as jnp
from jax import lax
from jax.experimental import pallas as pl
from jax.experimental.pallas import tpu as pltpu
from jax.experimental.pallas import tpu_sc as plsc

N = 10000
E = 320000
B = 256
OUT = 128
NP = 10240
CHUNK = 128
NCORE = 2
NSUB = 16
NTILE = NCORE * NSUB
NCHUNK = 80
TILE_E = NCHUNK * CHUNK
EPAD = NTILE * TILE_E
ZROWS = NP // NSUB
DEGW = 128
EPS = 1e-5


def _zero_acc(zeros_hbm, rows, acc, s):
    pltpu.sync_copy(zeros_hbm, rows)
    for t in range(ZROWS // CHUNK):
        pltpu.sync_copy(rows, acc.at[pl.ds(s * ZROWS + t * CHUNK, CHUNK)])


def _deg_body(dst_hbm, zeros_hbm, ones_hbm, out_hbm, idxd, rows, acc, sem):
    c = lax.axis_index("c")
    s = lax.axis_index("s")
    wid = c * NSUB + s
    _zero_acc(zeros_hbm, rows, acc, s)
    plsc.subcore_barrier()
    pltpu.sync_copy(ones_hbm, rows)
    pltpu.sync_copy(dst_hbm.at[pl.ds(wid * NCHUNK, NCHUNK)], idxd)

    def fire(j, carry):
        pltpu.async_copy(rows, acc.at[idxd.at[j]], sem, add=True)
        return carry

    lax.fori_loop(0, NCHUNK, fire, 0)

    def drain(j, carry):
        pltpu.make_async_copy(rows, acc.at[idxd.at[0]], sem).wait()
        return carry

    lax.fori_loop(0, NCHUNK, drain, 0)
    plsc.subcore_barrier()
    pltpu.sync_copy(acc.at[pl.ds(s * ZROWS, ZROWS)],
                    out_hbm.at[pl.ds(c * NP + s * ZROWS, ZROWS)])


HC = 16


def _agg_body(vals_hbm, src_hbm, dst_hbm, zeros_hbm, out_hbm,
              idxs, idxd, rows0, rows1, acc, semg0, semg1, sems0, sems1):
    c = lax.axis_index("c")
    s = lax.axis_index("s")
    wid = c * NSUB + s
    _zero_acc(zeros_hbm, rows0, acc, s)
    plsc.subcore_barrier()

    def gather(j, buf, sem):
        return pltpu.async_copy(vals_hbm.at[idxs.at[j]], buf, sem)

    def scatter(j, buf, sem):
        return pltpu.async_copy(buf, acc.at[idxd.at[j]], sem, add=True)

    for h in range(NCHUNK // HC):
        base = wid * NCHUNK + h * HC
        pltpu.sync_copy(src_hbm.at[pl.ds(base, HC)], idxs)
        pltpu.sync_copy(dst_hbm.at[pl.ds(base, HC)], idxd)
        gather(0, rows0, semg0)

        def pipe(i, carry):
            j0 = 2 * i
            j1 = j0 + 1
            pltpu.make_async_copy(vals_hbm.at[idxs.at[j0]], rows0, semg0).wait()

            @pl.when(i > 0)
            def _():
                pltpu.make_async_copy(rows1, acc.at[idxd.at[0]], sems1).wait()

            gather(j1, rows1, semg1)
            scatter(j0, rows0, sems0)
            pltpu.make_async_copy(vals_hbm.at[idxs.at[j1]], rows1, semg1).wait()

            @pl.when(j0 + 2 < HC)
            def _():
                pltpu.make_async_copy(rows0, acc.at[idxd.at[0]], sems0).wait()
                gather(j0 + 2, rows0, semg0)

            scatter(j1, rows1, sems1)
            return carry

        lax.fori_loop(0, HC // 2, pipe, 0)
        pltpu.make_async_copy(rows0, acc.at[idxd.at[0]], sems0).wait()
        pltpu.make_async_copy(rows1, acc.at[idxd.at[0]], sems1).wait()
    plsc.subcore_barrier()
    pltpu.sync_copy(acc.at[pl.ds(s * ZROWS, ZROWS)],
                    out_hbm.at[pl.ds(c * NP + s * ZROWS, ZROWS)])


@functools.cache
def _sc_kernels():
    mesh = plsc.VectorSubcoreMesh(core_axis_name="c", subcore_axis_name="s",
                                  num_cores=NCORE, num_subcores=NSUB)
    deg_k = pl.kernel(
        _deg_body,
        out_type=jax.ShapeDtypeStruct((NCORE * NP, DEGW), jnp.float32),
        mesh=mesh,
        scratch_types=[
            pltpu.VMEM((NCHUNK, CHUNK), jnp.int32),
            pltpu.VMEM((CHUNK, DEGW), jnp.float32),
            pltpu.VMEM_SHARED((NP, DEGW), jnp.float32),
            pltpu.SemaphoreType.DMA,
        ],
    )
    agg_k = pl.kernel(
        _agg_body,
        out_type=jax.ShapeDtypeStruct((NCORE * NP, OUT), jnp.float32),
        mesh=mesh,
        scratch_types=[
            pltpu.VMEM((HC, CHUNK), jnp.int32),
            pltpu.VMEM((HC, CHUNK), jnp.int32),
            pltpu.VMEM((CHUNK, OUT), jnp.float32),
            pltpu.VMEM((CHUNK, OUT), jnp.float32),
            pltpu.VMEM_SHARED((NP, OUT), jnp.float32),
            pltpu.SemaphoreType.DMA,
            pltpu.SemaphoreType.DMA,
            pltpu.SemaphoreType.DMA,
            pltpu.SemaphoreType.DMA,
        ],
    )
    return deg_k, agg_k


def _tc1_body(degp, xp, w1, hs1, dinvb):
    deg = degp[0, :, 0:1] + degp[1, :, 0:1] + 1.0
    db = jnp.broadcast_to(lax.rsqrt(deg), (NP, OUT))
    dinvb[...] = db
    h = jnp.dot(xp[...], w1[...], preferred_element_type=jnp.float32)
    hs1[...] = h * db


_tc1 = pl.pallas_call(
    _tc1_body,
    out_shape=[
        jax.ShapeDtypeStruct((NP, OUT), jnp.float32),
        jax.ShapeDtypeStruct((NP, OUT), jnp.float32),
    ],
)


def _bn_masked(a):
    rid = lax.broadcasted_iota(jnp.int32, (NP, 1), 0)
    m = rid < N
    am = jnp.where(m, a, 0.0)
    mu = jnp.sum(am, axis=0, keepdims=True) / N
    d = jnp.where(m, a - mu, 0.0)
    var = jnp.sum(d * d, axis=0, keepdims=True) / N
    return mu, var


def _tc2_body(sp, hsp, dinvb, b, g, be, w, out):
    db = dinvb[...]
    z = db * (sp[0] + sp[1] + hsp[...]) + b[...]
    a = jnp.maximum(z, 0.0)
    mu, var = _bn_masked(a)
    y = (a - mu) * lax.rsqrt(var + EPS) * g[...] + be[...]
    out[...] = jnp.dot(y, w[...], preferred_element_type=jnp.float32) * db


_tc2 = pl.pallas_call(
    _tc2_body,
    out_shape=jax.ShapeDtypeStruct((NP, OUT), jnp.float32),
)


def _tc3_body(sp, hsp, dinvb, b, g, be, batchc, gx, wc1, bc1, gc1, bec1,
              wc2, bc2, xdrug, xcell):
    z = dinvb[...] * (sp[0] + sp[1] + hsp[...]) + b[...]
    a = jnp.maximum(z, 0.0)
    mu, var = _bn_masked(a)
    y = (a - mu) * lax.rsqrt(var + EPS) * g[...] + be[...]
    rid = lax.broadcasted_iota(jnp.int32, (NP, 1), 0)
    ym = jnp.where(rid < N, y, -jnp.inf)
    bc = batchc[...]

    def seg(bi, carry):
        vals = jnp.where(bc == bi, ym, -jnp.inf)
        xdrug[pl.ds(bi, 1), :] = jnp.max(vals, axis=0, keepdims=True)
        return carry

    lax.fori_loop(0, B, seg, 0)

    t = jnp.tanh(jnp.dot(gx[...], wc1[...],
                         preferred_element_type=jnp.float32) + bc1[...])
    cmu = jnp.mean(t, axis=0, keepdims=True)
    cvar = jnp.mean((t - cmu) ** 2, axis=0, keepdims=True)
    yc = (t - cmu) * lax.rsqrt(cvar + EPS) * gc1[...] + bec1[...]
    xcell[...] = jnp.maximum(
        jnp.dot(yc, wc2[...], preferred_element_type=jnp.float32) + bc2[...],
        0.0)


_tc3 = pl.pallas_call(
    _tc3_body,
    out_shape=[
        jax.ShapeDtypeStruct((B, OUT), jnp.float32),
        jax.ShapeDtypeStruct((B, OUT), jnp.float32),
    ],
)


def kernel(drug_x, edge_index, batch, gexpr, W1, b1, g1, be1, W2, b2, g2, be2,
           Wc1, bc1, gc1, bec1, Wc2, bc2):
    src = edge_index[0].astype(jnp.int32)
    dst = edge_index[1].astype(jnp.int32)
    pad = jnp.full((EPAD - E,), N, jnp.int32)
    srcp = jnp.concatenate([src, pad]).reshape(NTILE * NCHUNK, CHUNK)
    dstp = jnp.concatenate([dst, pad]).reshape(NTILE * NCHUNK, CHUNK)
    xp = jnp.pad(drug_x, ((0, NP - N), (0, 0)))
    zeros_f = jnp.zeros((CHUNK, OUT), jnp.float32)
    zeros_d = jnp.zeros((CHUNK, DEGW), jnp.float32)
    ones_d = jnp.ones((CHUNK, DEGW), jnp.float32)

    deg_k, agg_k = _sc_kernels()
    degp = deg_k(dstp, zeros_d, ones_d).reshape(NCORE, NP, DEGW)
    hs1, dinvb = _tc1(degp, xp, W1)
    s1 = agg_k(hs1, srcp, dstp, zeros_f).reshape(NCORE, NP, OUT)
    hs2 = _tc2(s1, hs1, dinvb, b1.reshape(1, OUT), g1.reshape(1, OUT),
               be1.reshape(1, OUT), W2)
    s2 = agg_k(hs2, srcp, dstp, zeros_f).reshape(NCORE, NP, OUT)

    batchc = jnp.pad(batch.astype(jnp.int32), (0, NP - N)).reshape(NP, 1)
    gxp = jnp.pad(gexpr, ((0, 0), (0, 1024 - gexpr.shape[1])))
    wc1p = jnp.pad(Wc1, ((0, 1024 - Wc1.shape[0]), (0, 0)))
    x_drug, x_cell = _tc3(
        s2, hs2, dinvb, b2.reshape(1, OUT), g2.reshape(1, OUT),
        be2.reshape(1, OUT), batchc, gxp, wc1p, bc1.reshape(1, OUT),
        gc1.reshape(1, OUT), bec1.reshape(1, OUT), Wc2, bc2.reshape(1, OUT))
    return (x_drug, x_cell)


# speedup vs baseline: 1.3890x; 1.3890x over previous
import functools

import jax
import jax.numpy as jnp
from jax import lax
from jax.experimental import pallas as pl
from jax.experimental.pallas import tpu as pltpu
from jax.experimental.pallas import tpu_sc as plsc

N = 10000
E = 320000
B = 256
OUT = 128
NP = 10240
CHUNK = 128
NCORE = 2
NSUB = 16
NTILE = NCORE * NSUB
NCHUNK = 80
TILE_E = NCHUNK * CHUNK
EPAD = NTILE * TILE_E
ZROWS = NP // NSUB
DEGW = 128
EPS = 1e-5


def _zero_acc(zeros_hbm, rows, acc, s):
    nr = rows.shape[0]
    pltpu.sync_copy(zeros_hbm, rows)
    for t in range(ZROWS // nr):
        pltpu.sync_copy(rows, acc.at[pl.ds(s * ZROWS + t * nr, nr)])


def _deg_body(dst_hbm, zeros_hbm, ones_hbm, out_hbm, idxd, rows, acc, sem):
    c = lax.axis_index("c")
    s = lax.axis_index("s")
    wid = c * NSUB + s
    _zero_acc(zeros_hbm, rows, acc, s)
    plsc.subcore_barrier()
    pltpu.sync_copy(ones_hbm, rows)
    pltpu.sync_copy(dst_hbm.at[pl.ds(wid * NCHUNK, NCHUNK)], idxd)

    def fire(j, carry):
        pltpu.async_copy(rows, acc.at[idxd.at[j]], sem, add=True)
        return carry

    lax.fori_loop(0, NCHUNK, fire, 0)

    def drain(j, carry):
        pltpu.make_async_copy(rows, acc.at[idxd.at[0]], sem).wait()
        return carry

    lax.fori_loop(0, NCHUNK, drain, 0)
    plsc.subcore_barrier()
    pltpu.sync_copy(acc.at[pl.ds(s * ZROWS, ZROWS)],
                    out_hbm.at[pl.ds(c * NP + s * ZROWS, ZROWS)])


GC = 64
GCHUNK = EPAD // (NTILE * GC)
HC = 32
NBUF = 4


def _agg_body(vals_hbm, src_hbm, dst_hbm, zeros_hbm, out_hbm,
              idxs, idxd, r0, r1, r2, r3, acc,
              sg0, sg1, sg2, sg3, ss0, ss1, ss2, ss3):
    c = lax.axis_index("c")
    s = lax.axis_index("s")
    wid = c * NSUB + s
    _zero_acc(zeros_hbm, r0, acc, s)
    plsc.subcore_barrier()
    bufs = (r0, r1, r2, r3)
    sgs = (sg0, sg1, sg2, sg3)
    sss = (ss0, ss1, ss2, ss3)

    def gather(j, b):
        pltpu.async_copy(vals_hbm.at[idxs.at[j]], bufs[b], sgs[b])

    def wait_gather(j, b):
        pltpu.make_async_copy(vals_hbm.at[idxs.at[j]], bufs[b], sgs[b]).wait()

    def scatter(j, b):
        pltpu.async_copy(bufs[b], acc.at[idxd.at[j]], sss[b], add=True)

    def wait_scatter(b):
        pltpu.make_async_copy(bufs[b], acc.at[idxd.at[0]], sss[b]).wait()

    for h in range(GCHUNK // HC):
        base = wid * GCHUNK + h * HC
        pltpu.sync_copy(src_hbm.at[pl.ds(base, HC)], idxs)
        pltpu.sync_copy(dst_hbm.at[pl.ds(base, HC)], idxd)
        for b in range(NBUF):
            gather(b, b)

        def quad(q, carry):
            for b in range(NBUF):
                j = NBUF * q + b
                wait_gather(j, b)
                scatter(j, b)

                @pl.when(j + NBUF < HC)
                def _():
                    wait_scatter(b)
                    gather(j + NBUF, b)

            return carry

        lax.fori_loop(0, HC // NBUF, quad, 0)
        for b in range(NBUF):
            wait_scatter(b)
    plsc.subcore_barrier()
    pltpu.sync_copy(acc.at[pl.ds(s * ZROWS, ZROWS)],
                    out_hbm.at[pl.ds(c * NP + s * ZROWS, ZROWS)])


@functools.cache
def _sc_kernels():
    mesh = plsc.VectorSubcoreMesh(core_axis_name="c", subcore_axis_name="s",
                                  num_cores=NCORE, num_subcores=NSUB)
    deg_k = pl.kernel(
        _deg_body,
        out_type=jax.ShapeDtypeStruct((NCORE * NP, DEGW), jnp.float32),
        mesh=mesh,
        scratch_types=[
            pltpu.VMEM((NCHUNK, CHUNK), jnp.int32),
            pltpu.VMEM((CHUNK, DEGW), jnp.float32),
            pltpu.VMEM_SHARED((NP, DEGW), jnp.float32),
            pltpu.SemaphoreType.DMA,
        ],
    )
    agg_k = pl.kernel(
        _agg_body,
        out_type=jax.ShapeDtypeStruct((NCORE * NP, OUT), jnp.float32),
        mesh=mesh,
        scratch_types=(
            [pltpu.VMEM((HC, GC), jnp.int32)] * 2
            + [pltpu.VMEM((GC, OUT), jnp.float32)] * NBUF
            + [pltpu.VMEM_SHARED((NP, OUT), jnp.float32)]
            + [pltpu.SemaphoreType.DMA] * (2 * NBUF)
        ),
    )
    return deg_k, agg_k


def _tc1_body(degp, xp, w1, hs1, dinvb):
    deg = degp[0, :, 0:1] + degp[1, :, 0:1] + 1.0
    db = jnp.broadcast_to(lax.rsqrt(deg), (NP, OUT))
    dinvb[...] = db
    h = jnp.dot(xp[...], w1[...], preferred_element_type=jnp.float32)
    hs1[...] = h * db


_tc1 = pl.pallas_call(
    _tc1_body,
    out_shape=[
        jax.ShapeDtypeStruct((NP, OUT), jnp.float32),
        jax.ShapeDtypeStruct((NP, OUT), jnp.float32),
    ],
)


def _bn_masked(a):
    rid = lax.broadcasted_iota(jnp.int32, (NP, 1), 0)
    m = rid < N
    am = jnp.where(m, a, 0.0)
    mu = jnp.sum(am, axis=0, keepdims=True) / N
    d = jnp.where(m, a - mu, 0.0)
    var = jnp.sum(d * d, axis=0, keepdims=True) / N
    return mu, var


def _tc2_body(sp, hsp, dinvb, b, g, be, w, out):
    db = dinvb[...]
    z = db * (sp[0] + sp[1] + hsp[...]) + b[...]
    a = jnp.maximum(z, 0.0)
    mu, var = _bn_masked(a)
    y = (a - mu) * lax.rsqrt(var + EPS) * g[...] + be[...]
    out[...] = jnp.dot(y, w[...], preferred_element_type=jnp.float32) * db


_tc2 = pl.pallas_call(
    _tc2_body,
    out_shape=jax.ShapeDtypeStruct((NP, OUT), jnp.float32),
)


NP2 = NP + B
WIN = 256


def _tc3a_body(sp, hsp, dinvb, b, g, be, batchc, ym_out, cnt_out):
    z = dinvb[...] * (sp[0] + sp[1] + hsp[...]) + b[...]
    a = jnp.maximum(z, 0.0)
    mu, var = _bn_masked(a)
    y = (a - mu) * lax.rsqrt(var + EPS) * g[...] + be[...]
    rid = lax.broadcasted_iota(jnp.int32, (NP, 1), 0)
    ym = jnp.where(rid < N, y, -jnp.inf)
    ym_out[...] = jnp.concatenate(
        [ym, jnp.full((NP2 - NP, OUT), -jnp.inf, jnp.float32)], axis=0)
    bc = batchc[...].astype(jnp.int32)
    ib = lax.broadcasted_iota(jnp.int32, (1, 2 * B), 1)
    lt = jnp.where(bc < ib, 1.0, 0.0)
    ones = jnp.ones((1, NP), jnp.float32)
    cnt_out[...] = jnp.dot(ones, lt,
                           preferred_element_type=jnp.float32).astype(jnp.int32)


_tc3a = pl.pallas_call(
    _tc3a_body,
    out_shape=[
        jax.ShapeDtypeStruct((NP2, OUT), jnp.float32),
        jax.ShapeDtypeStruct((1, 2 * B), jnp.int32),
    ],
)


def _tc3b_body(cnt, ym, batche, gx, wc1, bc1, gc1, bec1, wc2, bc2,
               xdrug, xcell):
    def seg(bi, carry):
        o0 = cnt[bi]
        o1 = cnt[bi + 1]
        ob8 = (o0 // 8) * 8
        nwin = (o1 - ob8 + WIN - 1) // WIN

        def win(k, acc):
            st = pl.multiple_of(ob8 + k * WIN, 8)
            w = ym[pl.ds(st, WIN), :]
            bb = batche[pl.ds(st, WIN), :]
            vals = jnp.where(bb == bi, w, -jnp.inf)
            return jnp.maximum(acc, jnp.max(vals, axis=0, keepdims=True))

        acc = lax.fori_loop(0, nwin, win,
                            jnp.full((1, OUT), -jnp.inf, jnp.float32))
        xdrug[pl.ds(bi, 1), :] = acc
        return carry

    lax.fori_loop(0, B, seg, 0)

    t = jnp.tanh(jnp.dot(gx[...], wc1[...],
                         preferred_element_type=jnp.float32) + bc1[...])
    cmu = jnp.mean(t, axis=0, keepdims=True)
    cvar = jnp.mean((t - cmu) ** 2, axis=0, keepdims=True)
    yc = (t - cmu) * lax.rsqrt(cvar + EPS) * gc1[...] + bec1[...]
    xcell[...] = jnp.maximum(
        jnp.dot(yc, wc2[...], preferred_element_type=jnp.float32) + bc2[...],
        0.0)


_tc3b = pl.pallas_call(
    _tc3b_body,
    in_specs=[pl.BlockSpec(memory_space=pltpu.SMEM)]
    + [pl.BlockSpec(memory_space=pltpu.VMEM)] * 9,
    out_shape=[
        jax.ShapeDtypeStruct((B, OUT), jnp.float32),
        jax.ShapeDtypeStruct((B, OUT), jnp.float32),
    ],
)


def kernel(drug_x, edge_index, batch, gexpr, W1, b1, g1, be1, W2, b2, g2, be2,
           Wc1, bc1, gc1, bec1, Wc2, bc2):
    src = edge_index[0].astype(jnp.int32)
    dst = edge_index[1].astype(jnp.int32)
    pad = jnp.full((EPAD - E,), N, jnp.int32)
    srcf = jnp.concatenate([src, pad])
    dstf = jnp.concatenate([dst, pad])
    srcp = srcf.reshape(NTILE * GCHUNK, GC)
    dstp = dstf.reshape(NTILE * GCHUNK, GC)
    dstp128 = dstf.reshape(NTILE * NCHUNK, CHUNK)
    xp = jnp.pad(drug_x, ((0, NP - N), (0, 0)))
    zeros_f = jnp.zeros((GC, OUT), jnp.float32)
    zeros_d = jnp.zeros((CHUNK, DEGW), jnp.float32)
    ones_d = jnp.ones((CHUNK, DEGW), jnp.float32)

    deg_k, agg_k = _sc_kernels()
    degp = deg_k(dstp128, zeros_d, ones_d).reshape(NCORE, NP, DEGW)
    hs1, dinvb = _tc1(degp, xp, W1)
    s1 = agg_k(hs1, srcp, dstp, zeros_f).reshape(NCORE, NP, OUT)
    hs2 = _tc2(s1, hs1, dinvb, b1.reshape(1, OUT), g1.reshape(1, OUT),
               be1.reshape(1, OUT), W2)
    s2 = agg_k(hs2, srcp, dstp, zeros_f).reshape(NCORE, NP, OUT)

    batchc = jnp.pad(batch.astype(jnp.int32), (0, NP - N),
                     constant_values=B).reshape(NP, 1)
    batche = jnp.pad(batch.astype(jnp.int32), (0, NP2 - N),
                     constant_values=B).reshape(NP2, 1)
    gxp = jnp.pad(gexpr, ((0, 0), (0, 1024 - gexpr.shape[1])))
    wc1p = jnp.pad(Wc1, ((0, 1024 - Wc1.shape[0]), (0, 0)))
    ym, cnt = _tc3a(s2, hs2, dinvb, b2.reshape(1, OUT), g2.reshape(1, OUT),
                    be2.reshape(1, OUT), batchc)
    x_drug, x_cell = _tc3b(
        cnt.reshape(2 * B), ym, batche, gxp, wc1p, bc1.reshape(1, OUT),
        gc1.reshape(1, OUT), bec1.reshape(1, OUT), Wc2, bc2.reshape(1, OUT))
    return (x_drug, x_cell)


# speedup vs baseline: 1.3969x; 1.0057x over previous
import functools

import jax
import jax.numpy as jnp
from jax import lax
from jax.experimental import pallas as pl
from jax.experimental.pallas import tpu as pltpu
from jax.experimental.pallas import tpu_sc as plsc

N = 10000
E = 320000
B = 256
OUT = 128
NP = 10240
CHUNK = 128
NCORE = 2
NSUB = 16
NTILE = NCORE * NSUB
NCHUNK = 80
TILE_E = NCHUNK * CHUNK
EPAD = NTILE * TILE_E
ZROWS = NP // NSUB
DEGW = 128
EPS = 1e-5


def _zero_acc(zeros_hbm, rows, acc, s):
    nr = rows.shape[0]
    pltpu.sync_copy(zeros_hbm, rows)
    for t in range(ZROWS // nr):
        pltpu.sync_copy(rows, acc.at[pl.ds(s * ZROWS + t * nr, nr)])


def _deg_body(dst_hbm, zeros_hbm, ones_hbm, out_hbm, idxd, rows, acc, sem):
    c = lax.axis_index("c")
    s = lax.axis_index("s")
    wid = c * NSUB + s
    _zero_acc(zeros_hbm, rows, acc, s)
    plsc.subcore_barrier()
    pltpu.sync_copy(ones_hbm, rows)
    pltpu.sync_copy(dst_hbm.at[pl.ds(wid * NCHUNK, NCHUNK)], idxd)

    def fire(j, carry):
        pltpu.async_copy(rows, acc.at[idxd.at[j]], sem, add=True)
        return carry

    lax.fori_loop(0, NCHUNK, fire, 0)

    def drain(j, carry):
        pltpu.make_async_copy(rows, acc.at[idxd.at[0]], sem).wait()
        return carry

    lax.fori_loop(0, NCHUNK, drain, 0)
    plsc.subcore_barrier()
    pltpu.sync_copy(acc.at[pl.ds(s * ZROWS, ZROWS)],
                    out_hbm.at[pl.ds(c * NP + s * ZROWS, ZROWS)])


GC = 32
GCHUNK = EPAD // (NTILE * GC)
HC = 32
NBUF = 8


def _agg_body(vals_hbm, src_hbm, dst_hbm, zeros_hbm, out_hbm,
              idxs, idxd, r0, r1, r2, r3, r4, r5, r6, r7, acc,
              sg0, sg1, sg2, sg3, sg4, sg5, sg6, sg7,
              ss0, ss1, ss2, ss3, ss4, ss5, ss6, ss7):
    c = lax.axis_index("c")
    s = lax.axis_index("s")
    wid = c * NSUB + s
    _zero_acc(zeros_hbm, r0, acc, s)
    plsc.subcore_barrier()
    bufs = (r0, r1, r2, r3, r4, r5, r6, r7)
    sgs = (sg0, sg1, sg2, sg3, sg4, sg5, sg6, sg7)
    sss = (ss0, ss1, ss2, ss3, ss4, ss5, ss6, ss7)

    def gather(j, b):
        pltpu.async_copy(vals_hbm.at[idxs.at[j]], bufs[b], sgs[b])

    def wait_gather(j, b):
        pltpu.make_async_copy(vals_hbm.at[idxs.at[j]], bufs[b], sgs[b]).wait()

    def scatter(j, b):
        pltpu.async_copy(bufs[b], acc.at[idxd.at[j]], sss[b], add=True)

    def wait_scatter(b):
        pltpu.make_async_copy(bufs[b], acc.at[idxd.at[0]], sss[b]).wait()

    for h in range(GCHUNK // HC):
        base = wid * GCHUNK + h * HC
        pltpu.sync_copy(src_hbm.at[pl.ds(base, HC)], idxs)
        pltpu.sync_copy(dst_hbm.at[pl.ds(base, HC)], idxd)
        for b in range(NBUF):
            gather(b, b)

        def ring(q, carry):
            for b in range(NBUF):
                j = NBUF * q + b
                wait_gather(j, b)
                scatter(j, b)

                @pl.when(j + NBUF < HC)
                def _():
                    wait_scatter(b)
                    gather(j + NBUF, b)

            return carry

        lax.fori_loop(0, HC // NBUF, ring, 0)
        for b in range(NBUF):
            wait_scatter(b)
    plsc.subcore_barrier()
    pltpu.sync_copy(acc.at[pl.ds(s * ZROWS, ZROWS)],
                    out_hbm.at[pl.ds(c * NP + s * ZROWS, ZROWS)])


@functools.cache
def _sc_kernels():
    mesh = plsc.VectorSubcoreMesh(core_axis_name="c", subcore_axis_name="s",
                                  num_cores=NCORE, num_subcores=NSUB)
    deg_k = pl.kernel(
        _deg_body,
        out_type=jax.ShapeDtypeStruct((NCORE * NP, DEGW), jnp.float32),
        mesh=mesh,
        scratch_types=[
            pltpu.VMEM((NCHUNK, CHUNK), jnp.int32),
            pltpu.VMEM((CHUNK, DEGW), jnp.float32),
            pltpu.VMEM_SHARED((NP, DEGW), jnp.float32),
            pltpu.SemaphoreType.DMA,
        ],
    )
    agg_k = pl.kernel(
        _agg_body,
        out_type=jax.ShapeDtypeStruct((NCORE * NP, OUT), jnp.float32),
        mesh=mesh,
        scratch_types=(
            [pltpu.VMEM((HC, GC), jnp.int32)] * 2
            + [pltpu.VMEM((GC, OUT), jnp.float32)] * NBUF
            + [pltpu.VMEM_SHARED((NP, OUT), jnp.float32)]
            + [pltpu.SemaphoreType.DMA] * (2 * NBUF)
        ),
    )
    return deg_k, agg_k


def _tc1_body(degp, xp, w1, hs1, dinvb):
    deg = degp[0, :, 0:1] + degp[1, :, 0:1] + 1.0
    db = jnp.broadcast_to(lax.rsqrt(deg), (NP, OUT))
    dinvb[...] = db
    h = jnp.dot(xp[...], w1[...], preferred_element_type=jnp.float32)
    hs1[...] = h * db


_tc1 = pl.pallas_call(
    _tc1_body,
    out_shape=[
        jax.ShapeDtypeStruct((NP, OUT), jnp.float32),
        jax.ShapeDtypeStruct((NP, OUT), jnp.float32),
    ],
)


def _bn_masked(a):
    rid = lax.broadcasted_iota(jnp.int32, (NP, 1), 0)
    m = rid < N
    am = jnp.where(m, a, 0.0)
    mu = jnp.sum(am, axis=0, keepdims=True) / N
    d = jnp.where(m, a - mu, 0.0)
    var = jnp.sum(d * d, axis=0, keepdims=True) / N
    return mu, var


def _tc2_body(sp, hsp, dinvb, b, g, be, w, out):
    db = dinvb[...]
    z = db * (sp[0] + sp[1] + hsp[...]) + b[...]
    a = jnp.maximum(z, 0.0)
    mu, var = _bn_masked(a)
    y = (a - mu) * lax.rsqrt(var + EPS) * g[...] + be[...]
    out[...] = jnp.dot(y, w[...], preferred_element_type=jnp.float32) * db


_tc2 = pl.pallas_call(
    _tc2_body,
    out_shape=jax.ShapeDtypeStruct((NP, OUT), jnp.float32),
)


NP2 = NP + B
WIN = 256


def _tc3a_body(sp, hsp, dinvb, b, g, be, batchc, ym_out, cnt_out):
    z = dinvb[...] * (sp[0] + sp[1] + hsp[...]) + b[...]
    a = jnp.maximum(z, 0.0)
    mu, var = _bn_masked(a)
    y = (a - mu) * lax.rsqrt(var + EPS) * g[...] + be[...]
    rid = lax.broadcasted_iota(jnp.int32, (NP, 1), 0)
    ym = jnp.where(rid < N, y, -jnp.inf)
    ym_out[...] = jnp.concatenate(
        [ym, jnp.full((NP2 - NP, OUT), -jnp.inf, jnp.float32)], axis=0)
    bc = batchc[...].astype(jnp.int32)
    ib = lax.broadcasted_iota(jnp.int32, (1, 2 * B), 1)
    lt = jnp.where(bc < ib, 1.0, 0.0)
    ones = jnp.ones((1, NP), jnp.float32)
    cnt_out[...] = jnp.dot(ones, lt,
                           preferred_element_type=jnp.float32).astype(jnp.int32)


_tc3a = pl.pallas_call(
    _tc3a_body,
    out_shape=[
        jax.ShapeDtypeStruct((NP2, OUT), jnp.float32),
        jax.ShapeDtypeStruct((1, 2 * B), jnp.int32),
    ],
)


def _tc3b_body(cnt, ym, batche, gx, wc1, bc1, gc1, bec1, wc2, bc2,
               xdrug, xcell):
    def seg(bi, carry):
        o0 = cnt[bi]
        o1 = cnt[bi + 1]
        ob8 = (o0 // 8) * 8
        nwin = (o1 - ob8 + WIN - 1) // WIN

        def win(k, acc):
            st = pl.multiple_of(ob8 + k * WIN, 8)
            w = ym[pl.ds(st, WIN), :]
            bb = batche[pl.ds(st, WIN), :]
            vals = jnp.where(bb == bi, w, -jnp.inf)
            return jnp.maximum(acc, jnp.max(vals, axis=0, keepdims=True))

        acc = lax.fori_loop(0, nwin, win,
                            jnp.full((1, OUT), -jnp.inf, jnp.float32))
        xdrug[pl.ds(bi, 1), :] = acc
        return carry

    lax.fori_loop(0, B, seg, 0)

    t = jnp.tanh(jnp.dot(gx[...], wc1[...],
                         preferred_element_type=jnp.float32) + bc1[...])
    cmu = jnp.mean(t, axis=0, keepdims=True)
    cvar = jnp.mean((t - cmu) ** 2, axis=0, keepdims=True)
    yc = (t - cmu) * lax.rsqrt(cvar + EPS) * gc1[...] + bec1[...]
    xcell[...] = jnp.maximum(
        jnp.dot(yc, wc2[...], preferred_element_type=jnp.float32) + bc2[...],
        0.0)


_tc3b = pl.pallas_call(
    _tc3b_body,
    in_specs=[pl.BlockSpec(memory_space=pltpu.SMEM)]
    + [pl.BlockSpec(memory_space=pltpu.VMEM)] * 9,
    out_shape=[
        jax.ShapeDtypeStruct((B, OUT), jnp.float32),
        jax.ShapeDtypeStruct((B, OUT), jnp.float32),
    ],
)


def kernel(drug_x, edge_index, batch, gexpr, W1, b1, g1, be1, W2, b2, g2, be2,
           Wc1, bc1, gc1, bec1, Wc2, bc2):
    src = edge_index[0].astype(jnp.int32)
    dst = edge_index[1].astype(jnp.int32)
    pad = jnp.full((EPAD - E,), N, jnp.int32)
    srcf = jnp.concatenate([src, pad])
    dstf = jnp.concatenate([dst, pad])
    srcp = srcf.reshape(NTILE * GCHUNK, GC)
    dstp = dstf.reshape(NTILE * GCHUNK, GC)
    dstp128 = dstf.reshape(NTILE * NCHUNK, CHUNK)
    xp = jnp.pad(drug_x, ((0, NP - N), (0, 0)))
    zeros_f = jnp.zeros((GC, OUT), jnp.float32)
    zeros_d = jnp.zeros((CHUNK, DEGW), jnp.float32)
    ones_d = jnp.ones((CHUNK, DEGW), jnp.float32)

    deg_k, agg_k = _sc_kernels()
    degp = deg_k(dstp128, zeros_d, ones_d).reshape(NCORE, NP, DEGW)
    hs1, dinvb = _tc1(degp, xp, W1)
    s1 = agg_k(hs1, srcp, dstp, zeros_f).reshape(NCORE, NP, OUT)
    hs2 = _tc2(s1, hs1, dinvb, b1.reshape(1, OUT), g1.reshape(1, OUT),
               be1.reshape(1, OUT), W2)
    s2 = agg_k(hs2, srcp, dstp, zeros_f).reshape(NCORE, NP, OUT)

    batchc = jnp.pad(batch.astype(jnp.int32), (0, NP - N),
                     constant_values=B).reshape(NP, 1)
    batche = jnp.pad(batch.astype(jnp.int32), (0, NP2 - N),
                     constant_values=B).reshape(NP2, 1)
    gxp = jnp.pad(gexpr, ((0, 0), (0, 1024 - gexpr.shape[1])))
    wc1p = jnp.pad(Wc1, ((0, 1024 - Wc1.shape[0]), (0, 0)))
    ym, cnt = _tc3a(s2, hs2, dinvb, b2.reshape(1, OUT), g2.reshape(1, OUT),
                    be2.reshape(1, OUT), batchc)
    x_drug, x_cell = _tc3b(
        cnt.reshape(2 * B), ym, batche, gxp, wc1p, bc1.reshape(1, OUT),
        gc1.reshape(1, OUT), bec1.reshape(1, OUT), Wc2, bc2.reshape(1, OUT))
    return (x_drug, x_cell)
